# baseline (speedup 1.0000x reference)
import functools

import jax
import jax.numpy as jnp
from jax import lax
from jax.experimental import pallas as pl
from jax.experimental.pallas import tpu as pltpu
from jax.experimental.pallas import tpu_sc as plsc

HIDDEN = 64
NUM_CORES = 2
NUM_SUBCORES = 16
NW = NUM_CORES * NUM_SUBCORES

IDX_MINOR = 128
CHUNK_ROWS = 8
CHUNK_TOK = CHUNK_ROWS * IDX_MINOR


def _gather_sc(table, idx2d, ntok):
    tok_per_w = ntok // NW
    chunks = tok_per_w // CHUNK_TOK
    rows_per_w = tok_per_w // IDX_MINOR

    mesh = plsc.VectorSubcoreMesh(core_axis_name="c", subcore_axis_name="s")

    @functools.partial(
        pl.kernel,
        mesh=mesh,
        out_type=jax.ShapeDtypeStruct((ntok, HIDDEN), jnp.float32),
        compiler_params=pltpu.CompilerParams(use_tc_tiling_on_sc=False),
        scratch_types=[
            pltpu.VMEM((CHUNK_ROWS, IDX_MINOR), jnp.int32),
            pltpu.VMEM((CHUNK_TOK, HIDDEN), jnp.float32),
            pltpu.SemaphoreType.DMA,
        ],
    )
    def k(table_hbm, idx_hbm, out_hbm, idx_v, rows_v, sem):
        wid = lax.axis_index("s") * NUM_CORES + lax.axis_index("c")
        row_base = wid * rows_per_w
        tok_base = wid * tok_per_w

        def body(i, carry):
            pltpu.sync_copy(idx_hbm.at[pl.ds(row_base + i * CHUNK_ROWS,
                                             CHUNK_ROWS)], idx_v)
            copies = [
                pltpu.async_copy(
                    table_hbm.at[idx_v.at[j]],
                    rows_v.at[pl.ds(j * IDX_MINOR, IDX_MINOR)],
                    sem,
                )
                for j in range(CHUNK_ROWS)
            ]
            for c in copies:
                c.wait()
            pltpu.sync_copy(rows_v,
                            out_hbm.at[pl.ds(tok_base + i * CHUNK_TOK,
                                             CHUNK_TOK)])
            return carry

        lax.fori_loop(0, chunks, body, 0)

    return k(table, idx2d)


def _project_tc(x, w, b2d, ntok):
    rows = 8192
    grid = ntok // rows

    def body(x_ref, w_ref, b_ref, o_ref):
        o_ref[...] = (
            jnp.dot(x_ref[...], w_ref[...], preferred_element_type=jnp.float32)
            + b_ref[...]
        )

    return pl.pallas_call(
        body,
        grid=(grid,),
        in_specs=[
            pl.BlockSpec((rows, HIDDEN), lambda i: (i, 0)),
            pl.BlockSpec((HIDDEN, HIDDEN), lambda i: (0, 0)),
            pl.BlockSpec((1, HIDDEN), lambda i: (0, 0)),
        ],
        out_specs=pl.BlockSpec((rows, HIDDEN), lambda i: (i, 0)),
        out_shape=jax.ShapeDtypeStruct((ntok, HIDDEN), jnp.float32),
    )(x, w, b2d)


def kernel(input_ids, emb_table, W, b):
    batch, hist = input_ids.shape
    ntok = batch * hist
    idx2d = input_ids.reshape(ntok // IDX_MINOR, IDX_MINOR).astype(jnp.int32)
    gathered = _gather_sc(emb_table, idx2d, ntok)
    out = _project_tc(gathered, W, b.reshape(1, HIDDEN), ntok)
    return out.reshape(batch, hist, HIDDEN)


# speedup vs baseline: 1.6733x; 1.6733x over previous
import functools

import jax
import jax.numpy as jnp
from jax import lax
from jax.experimental import pallas as pl
from jax.experimental.pallas import tpu as pltpu
from jax.experimental.pallas import tpu_sc as plsc

HIDDEN = 64
NUM_CORES = 2
NUM_SUBCORES = 16
NW = NUM_CORES * NUM_SUBCORES

IDX_MINOR = 128
ROWS_PER_DMA = 128
DMAS_PER_CHUNK = 5
CHUNK_TOK = ROWS_PER_DMA * DMAS_PER_CHUNK


def _project_pack_tc(tt, w, b2d, vocab):
    vb = 8192
    grid = pl.cdiv(vocab, vb)

    def body(x_ref, w_ref, b_ref, o_ref):
        y = lax.dot_general(x_ref[...], w_ref[...], (((0,), (0,)), ((), ())),
                            preferred_element_type=jnp.float32) + b_ref[...]
        o_ref[:, 0:HIDDEN] = y

    return pl.pallas_call(
        body,
        grid=(grid,),
        in_specs=[
            pl.BlockSpec((HIDDEN, vb), lambda i: (0, i)),
            pl.BlockSpec((HIDDEN, HIDDEN), lambda i: (0, 0)),
            pl.BlockSpec((1, HIDDEN), lambda i: (0, 0)),
        ],
        out_specs=pl.BlockSpec((vb, 2 * HIDDEN), lambda i: (i, 0)),
        out_shape=jax.ShapeDtypeStruct((vocab, 2 * HIDDEN), jnp.float32),
    )(tt, w, b2d)


def _gather_sc(table, idx2d, ntok):
    tok_per_w = ntok // NW
    rows_per_w = tok_per_w // IDX_MINOR
    chunks = tok_per_w // CHUNK_TOK

    mesh = plsc.VectorSubcoreMesh(core_axis_name="c", subcore_axis_name="s")

    @functools.partial(
        pl.kernel,
        mesh=mesh,
        out_type=jax.ShapeDtypeStruct((ntok, HIDDEN), jnp.float32),
        compiler_params=pltpu.CompilerParams(use_tc_tiling_on_sc=False),
        scratch_types=[
            pltpu.VMEM((rows_per_w, IDX_MINOR), jnp.int32),
            pltpu.VMEM((CHUNK_TOK, HIDDEN), jnp.float32),
            pltpu.VMEM((CHUNK_TOK, HIDDEN), jnp.float32),
            pltpu.SemaphoreType.DMA,
            pltpu.SemaphoreType.DMA,
            pltpu.SemaphoreType.DMA,
            pltpu.SemaphoreType.DMA,
        ],
    )
    def k(tbl, idxh, outh, idx_v, buf0, buf1, gsem0, gsem1, wsem0, wsem1):
        wid = lax.axis_index("s") * NUM_CORES + lax.axis_index("c")
        rbase = wid * rows_per_w
        tbase = wid * tok_per_w
        bufs = (buf0, buf1)
        gsems = (gsem0, gsem1)
        wsems = (wsem0, wsem1)

        pltpu.sync_copy(idxh.at[pl.ds(rbase, rows_per_w)], idx_v)

        def g_desc(g, j, p):
            return pltpu.make_async_copy(
                tbl.at[idx_v.at[g * DMAS_PER_CHUNK + j]],
                bufs[p].at[pl.ds(j * ROWS_PER_DMA, ROWS_PER_DMA)],
                gsems[p])

        def w_desc(g, p):
            return pltpu.make_async_copy(
                bufs[p], outh.at[pl.ds(tbase + g * CHUNK_TOK, CHUNK_TOK)],
                wsems[p])

        def fire(g, p):
            for j in range(DMAS_PER_CHUNK):
                g_desc(g, j, p).start()

        def wait_g(g, p):
            for j in range(DMAS_PER_CHUNK):
                g_desc(g, j, p).wait()

        fire(0, 0)

        def body(t, carry):
            i = 2 * t

            @pl.when(i > 0)
            def _():
                w_desc(i - 1, 1).wait()

            fire(i + 1, 1)
            wait_g(i, 0)
            w_desc(i, 0).start()
            wait_g(i + 1, 1)
            w_desc(i + 1, 1).start()

            @pl.when(i + 2 < chunks)
            def _():
                w_desc(i, 0).wait()
                fire(i + 2, 0)

            return carry

        lax.fori_loop(0, chunks // 2, body, 0)
        w_desc(chunks - 2, 0).wait()
        w_desc(chunks - 1, 1).wait()

    return k(table, idx2d)


def _unpack_transpose_tc(g128, hist, batch):
    cb = batch // 2

    def body(x_ref, o_ref):
        x = x_ref[...]
        hh = lax.broadcasted_iota(jnp.int32, (HIDDEN, 2 * HIDDEN), 0)
        mm = lax.broadcasted_iota(jnp.int32, (HIDDEN, 2 * HIDDEN), 1)
        i_lo = (mm == hh).astype(jnp.float32)
        i_hi = (mm == hh + HIDDEN).astype(jnp.float32)
        o_ref[0, :, 0:cb] = lax.dot_general(
            i_lo, x, (((1,), (1,)), ((), ())),
            preferred_element_type=jnp.float32)
        o_ref[0, :, cb:batch] = lax.dot_general(
            i_hi, x, (((1,), (1,)), ((), ())),
            preferred_element_type=jnp.float32)

    return pl.pallas_call(
        body,
        grid=(hist,),
        in_specs=[pl.BlockSpec((cb, 2 * HIDDEN), lambda i: (i, 0))],
        out_specs=pl.BlockSpec((1, HIDDEN, batch), lambda i: (i, 0, 0)),
        out_shape=jax.ShapeDtypeStruct((hist, HIDDEN, batch), jnp.float32),
    )(g128)


def kernel(input_ids, emb_table, W, b):
    batch, hist = input_ids.shape
    vocab = emb_table.shape[0]
    ntok = batch * hist

    tt = jnp.transpose(emb_table)
    p2 = _project_pack_tc(tt, W, b.reshape(1, HIDDEN), vocab)
    pv = p2.reshape(2 * vocab, HIDDEN)

    ids_t = jnp.transpose(input_ids).astype(jnp.int32)
    ids_p = ids_t.reshape(hist, 2, batch // 2).transpose(0, 2, 1)
    idx2d = (2 * ids_p).reshape(ntok // IDX_MINOR, IDX_MINOR)

    g = _gather_sc(pv, idx2d, ntok)
    g128 = g.reshape(ntok // 2, 2 * HIDDEN)

    out_t = _unpack_transpose_tc(g128, hist, batch)
    return jnp.transpose(out_t, (2, 0, 1))


# speedup vs baseline: 2.0860x; 1.2466x over previous
import functools

import jax
import jax.numpy as jnp
from jax import lax
from jax.experimental import pallas as pl
from jax.experimental.pallas import tpu as pltpu
from jax.experimental.pallas import tpu_sc as plsc

HIDDEN = 64
NUM_CORES = 2
NUM_SUBCORES = 16
NW = NUM_CORES * NUM_SUBCORES

IDX_MINOR = 128
ROWS_PER_DMA = 128
DMAS_PER_CHUNK = 4
CHUNK_TOK = ROWS_PER_DMA * DMAS_PER_CHUNK

VB = 8192
SPLIT = 61 * VB


def _project_pack_tc(tt, w, b2d, vocab):
    nlo = 62

    def body(x_ref, w_ref, b_ref, o_ref):
        s = pl.program_id(1)
        y = lax.dot_general(x_ref[...], w_ref[...], (((0,), (0,)), ((), ())),
                            preferred_element_type=jnp.float32) + b_ref[...]

        @pl.when(s == 0)
        def _():
            o_ref[:, 0:HIDDEN] = y

        @pl.when(s == 1)
        def _():
            o_ref[:, HIDDEN:2 * HIDDEN] = y

    return pl.pallas_call(
        body,
        grid=(nlo, 2),
        in_specs=[
            pl.BlockSpec((HIDDEN, VB), lambda i, s: (0, i + 61 * s)),
            pl.BlockSpec((HIDDEN, HIDDEN), lambda i, s: (0, 0)),
            pl.BlockSpec((1, HIDDEN), lambda i, s: (0, 0)),
        ],
        out_specs=pl.BlockSpec((VB, 2 * HIDDEN), lambda i, s: (i, 0)),
        out_shape=jax.ShapeDtypeStruct((vocab - SPLIT, 2 * HIDDEN),
                                       jnp.float32),
    )(tt, w, b2d)


def _gather_scatter_sc(table, idx2d, ntok, batch):
    tok_per_w = ntok // NW
    rows_per_w = tok_per_w // IDX_MINOR
    chunks = tok_per_w // CHUNK_TOK
    halfb = batch // 2

    mesh = plsc.VectorSubcoreMesh(core_axis_name="c", subcore_axis_name="s")

    @functools.partial(
        pl.kernel,
        mesh=mesh,
        out_type=jax.ShapeDtypeStruct((ntok, HIDDEN), jnp.float32),
        compiler_params=pltpu.CompilerParams(use_tc_tiling_on_sc=False),
        scratch_types=[
            pltpu.VMEM((rows_per_w, IDX_MINOR), jnp.int32),
            pltpu.VMEM((CHUNK_TOK, HIDDEN), jnp.float32),
            pltpu.VMEM((CHUNK_TOK, HIDDEN), jnp.float32),
            pltpu.VMEM((DMAS_PER_CHUNK, IDX_MINOR), jnp.int32),
            pltpu.VMEM((DMAS_PER_CHUNK, IDX_MINOR), jnp.int32),
            pltpu.SemaphoreType.DMA,
            pltpu.SemaphoreType.DMA,
            pltpu.SemaphoreType.DMA,
            pltpu.SemaphoreType.DMA,
        ],
    )
    def k(tbl, idxh, outh, idx_v, buf0, buf1, dst0, dst1,
          gsem0, gsem1, wsem0, wsem1):
        wid = lax.axis_index("s") * NUM_CORES + lax.axis_index("c")
        rbase = wid * rows_per_w
        tbase = wid * tok_per_w
        bufs = (buf0, buf1)
        dsts = (dst0, dst1)
        gsems = (gsem0, gsem1)
        wsems = (wsem0, wsem1)

        pltpu.sync_copy(idxh.at[pl.ds(rbase, rows_per_w)], idx_v)

        iota16 = lax.iota(jnp.int32, 16)

        def g_desc(g, j, p):
            return pltpu.make_async_copy(
                tbl.at[idx_v.at[g * DMAS_PER_CHUNK + j]],
                bufs[p].at[pl.ds(j * ROWS_PER_DMA, ROWS_PER_DMA)],
                gsems[p])

        def w_desc(g, j, p):
            del g
            return pltpu.make_async_copy(
                bufs[p].at[pl.ds(j * ROWS_PER_DMA, ROWS_PER_DMA)],
                outh.at[dsts[p].at[j]],
                wsems[p])

        def fire(g, p):
            for j in range(DMAS_PER_CHUNK):
                g_desc(g, j, p).start()

        def wait_g(g, p):
            for j in range(DMAS_PER_CHUNK):
                g_desc(g, j, p).wait()

        def wb_start(g, p):
            for j in range(DMAS_PER_CHUNK):
                t0 = tbase + g * CHUNK_TOK + j * ROWS_PER_DMA
                m = lax.rem(t0, batch)
                base = (t0 - m) + 2 * lax.rem(m, halfb) + lax.div(m, halfb)
                for q in range(IDX_MINOR // 16):
                    dsts[p][j, pl.ds(q * 16, 16)] = (
                        iota16 * 2 + (base + 32 * q))
            for j in range(DMAS_PER_CHUNK):
                w_desc(g, j, p).start()

        def wb_wait(g, p):
            for j in range(DMAS_PER_CHUNK):
                w_desc(g, j, p).wait()

        fire(0, 0)

        def body(t, carry):
            i = 2 * t

            @pl.when(i > 0)
            def _():
                wb_wait(i - 1, 1)

            fire(i + 1, 1)
            wait_g(i, 0)
            wb_start(i, 0)
            wait_g(i + 1, 1)
            wb_start(i + 1, 1)

            @pl.when(i + 2 < chunks)
            def _():
                wb_wait(i, 0)
                fire(i + 2, 0)

            return carry

        lax.fori_loop(0, chunks // 2, body, 0)
        wb_wait(chunks - 2, 0)
        wb_wait(chunks - 1, 1)

    return k(table, idx2d)


def _unpack_transpose_tc(g128, hist, batch):
    cb = batch // 2

    def body(x_ref, o_ref):
        x = x_ref[...]
        hh = lax.broadcasted_iota(jnp.int32, (HIDDEN, 2 * HIDDEN), 0)
        mm = lax.broadcasted_iota(jnp.int32, (HIDDEN, 2 * HIDDEN), 1)
        i_lo = (mm == hh).astype(jnp.float32)
        i_hi = (mm == hh + HIDDEN).astype(jnp.float32)
        o_ref[0, :, 0:cb] = lax.dot_general(
            i_lo, x, (((1,), (1,)), ((), ())),
            preferred_element_type=jnp.float32)
        o_ref[0, :, cb:batch] = lax.dot_general(
            i_hi, x, (((1,), (1,)), ((), ())),
            preferred_element_type=jnp.float32)

    return pl.pallas_call(
        body,
        grid=(hist,),
        in_specs=[pl.BlockSpec((cb, 2 * HIDDEN), lambda i: (i, 0))],
        out_specs=pl.BlockSpec((1, HIDDEN, batch), lambda i: (i, 0, 0)),
        out_shape=jax.ShapeDtypeStruct((hist, HIDDEN, batch), jnp.float32),
    )(g128)


def kernel(input_ids, emb_table, W, b):
    batch, hist = input_ids.shape
    vocab = emb_table.shape[0]
    ntok = batch * hist

    tt = jnp.transpose(emb_table)
    p2 = _project_pack_tc(tt, W, b.reshape(1, HIDDEN), vocab)
    pv = p2.reshape(2 * (vocab - SPLIT), HIDDEN)

    ids_t = jnp.transpose(input_ids).astype(jnp.int32)
    ids_r = jnp.where(ids_t < SPLIT, 2 * ids_t, 2 * (ids_t - SPLIT) + 1)
    idx2d = ids_r.reshape(ntok // IDX_MINOR, IDX_MINOR)

    g = _gather_scatter_sc(pv, idx2d, ntok, batch)
    g128 = g.reshape(ntok // 2, 2 * HIDDEN)

    out_t = _unpack_transpose_tc(g128, hist, batch)
    return jnp.transpose(out_t, (2, 0, 1))


# speedup vs baseline: 2.4622x; 1.1803x over previous
import functools

import jax
import jax.numpy as jnp
from jax import lax
from jax.experimental import pallas as pl
from jax.experimental.pallas import tpu as pltpu
from jax.experimental.pallas import tpu_sc as plsc

HIDDEN = 64
NUM_CORES = 2
NUM_SUBCORES = 16
NW = NUM_CORES * NUM_SUBCORES

IDX_MINOR = 128
ROWS_PER_DMA = 128
DMAS_PER_CHUNK = 4
CHUNK_TOK = ROWS_PER_DMA * DMAS_PER_CHUNK

VB = 16384
NHI = 30
SPLIT = NHI * VB


def _project_pack_tc(tt, w, b2d, vocab):
    rows = vocab - SPLIT
    grid = pl.cdiv(rows, VB)

    def body(lo_ref, hi_ref, w_ref, b_ref, o_ref):
        wv = w_ref[...]
        bv = b_ref[...]
        o_ref[:, 0:HIDDEN] = lax.dot_general(
            lo_ref[...], wv, (((0,), (0,)), ((), ())),
            preferred_element_type=jnp.float32) + bv
        o_ref[:, HIDDEN:2 * HIDDEN] = lax.dot_general(
            hi_ref[...], wv, (((0,), (0,)), ((), ())),
            preferred_element_type=jnp.float32) + bv

    return pl.pallas_call(
        body,
        grid=(grid,),
        in_specs=[
            pl.BlockSpec((HIDDEN, VB), lambda i: (0, i)),
            pl.BlockSpec((HIDDEN, VB), lambda i: (0, i + NHI)),
            pl.BlockSpec((HIDDEN, HIDDEN), lambda i: (0, 0)),
            pl.BlockSpec((1, HIDDEN), lambda i: (0, 0)),
        ],
        out_specs=pl.BlockSpec((VB, 2 * HIDDEN), lambda i: (i, 0)),
        out_shape=jax.ShapeDtypeStruct((rows, 2 * HIDDEN), jnp.float32),
    )(tt, tt, w, b2d)


def _gather_scatter_sc(table, idx2d, ntok, batch):
    tok_per_w = ntok // NW
    rows_per_w = tok_per_w // IDX_MINOR
    chunks = tok_per_w // CHUNK_TOK
    halfb = batch // 2

    mesh = plsc.VectorSubcoreMesh(core_axis_name="c", subcore_axis_name="s")

    @functools.partial(
        pl.kernel,
        mesh=mesh,
        out_type=jax.ShapeDtypeStruct((ntok, HIDDEN), jnp.float32),
        compiler_params=pltpu.CompilerParams(use_tc_tiling_on_sc=False),
        scratch_types=[
            pltpu.VMEM((rows_per_w, IDX_MINOR), jnp.int32),
            pltpu.VMEM((CHUNK_TOK, HIDDEN), jnp.float32),
            pltpu.VMEM((CHUNK_TOK, HIDDEN), jnp.float32),
            pltpu.VMEM((DMAS_PER_CHUNK, IDX_MINOR), jnp.int32),
            pltpu.VMEM((DMAS_PER_CHUNK, IDX_MINOR), jnp.int32),
            pltpu.SemaphoreType.DMA,
            pltpu.SemaphoreType.DMA,
            pltpu.SemaphoreType.DMA,
            pltpu.SemaphoreType.DMA,
        ],
    )
    def k(tbl, idxh, outh, idx_v, buf0, buf1, dst0, dst1,
          gsem0, gsem1, wsem0, wsem1):
        wid = lax.axis_index("s") * NUM_CORES + lax.axis_index("c")
        rbase = wid * rows_per_w
        tbase = wid * tok_per_w
        bufs = (buf0, buf1)
        dsts = (dst0, dst1)
        gsems = (gsem0, gsem1)
        wsems = (wsem0, wsem1)

        pltpu.sync_copy(idxh.at[pl.ds(rbase, rows_per_w)], idx_v)

        iota16 = lax.iota(jnp.int32, 16)

        def g_desc(g, j, p):
            return pltpu.make_async_copy(
                tbl.at[idx_v.at[g * DMAS_PER_CHUNK + j]],
                bufs[p].at[pl.ds(j * ROWS_PER_DMA, ROWS_PER_DMA)],
                gsems[p])

        def w_desc(g, j, p):
            del g
            return pltpu.make_async_copy(
                bufs[p].at[pl.ds(j * ROWS_PER_DMA, ROWS_PER_DMA)],
                outh.at[dsts[p].at[j]],
                wsems[p])

        def fire(g, p):
            for j in range(DMAS_PER_CHUNK):
                g_desc(g, j, p).start()

        def wait_g(g, p):
            for j in range(DMAS_PER_CHUNK):
                g_desc(g, j, p).wait()

        def wb_start(g, p):
            for j in range(DMAS_PER_CHUNK):
                t0 = tbase + g * CHUNK_TOK + j * ROWS_PER_DMA
                m = lax.rem(t0, batch)
                base = (t0 - m) + 2 * lax.rem(m, halfb) + lax.div(m, halfb)
                for q in range(IDX_MINOR // 16):
                    dsts[p][j, pl.ds(q * 16, 16)] = (
                        iota16 * 2 + (base + 32 * q))
            for j in range(DMAS_PER_CHUNK):
                w_desc(g, j, p).start()

        def wb_wait(g, p):
            for j in range(DMAS_PER_CHUNK):
                w_desc(g, j, p).wait()

        fire(0, 0)

        def body(t, carry):
            i = 2 * t

            @pl.when(i > 0)
            def _():
                wb_wait(i - 1, 1)

            fire(i + 1, 1)
            wait_g(i, 0)
            wb_start(i, 0)
            wait_g(i + 1, 1)
            wb_start(i + 1, 1)

            @pl.when(i + 2 < chunks)
            def _():
                wb_wait(i, 0)
                fire(i + 2, 0)

            return carry

        lax.fori_loop(0, chunks // 2, body, 0)
        wb_wait(chunks - 2, 0)
        wb_wait(chunks - 1, 1)

    return k(table, idx2d)


def _unpack_transpose_tc(g128, hist, batch):
    cb = batch // 2

    def body(x_ref, o_ref):
        x = x_ref[...]
        hh = lax.broadcasted_iota(jnp.int32, (HIDDEN, 2 * HIDDEN), 0)
        mm = lax.broadcasted_iota(jnp.int32, (HIDDEN, 2 * HIDDEN), 1)
        i_lo = (mm == hh).astype(jnp.float32)
        i_hi = (mm == hh + HIDDEN).astype(jnp.float32)
        o_ref[0, :, 0:cb] = lax.dot_general(
            i_lo, x, (((1,), (1,)), ((), ())),
            preferred_element_type=jnp.float32)
        o_ref[0, :, cb:batch] = lax.dot_general(
            i_hi, x, (((1,), (1,)), ((), ())),
            preferred_element_type=jnp.float32)

    return pl.pallas_call(
        body,
        grid=(hist,),
        in_specs=[pl.BlockSpec((cb, 2 * HIDDEN), lambda i: (i, 0))],
        out_specs=pl.BlockSpec((1, HIDDEN, batch), lambda i: (i, 0, 0)),
        out_shape=jax.ShapeDtypeStruct((hist, HIDDEN, batch), jnp.float32),
    )(g128)


def kernel(input_ids, emb_table, W, b):
    batch, hist = input_ids.shape
    vocab = emb_table.shape[0]
    ntok = batch * hist

    tt = jnp.transpose(emb_table)
    p2 = _project_pack_tc(tt, W, b.reshape(1, HIDDEN), vocab)
    pv = p2.reshape(2 * (vocab - SPLIT), HIDDEN)

    ids_t = jnp.transpose(input_ids).astype(jnp.int32)
    ids_r = jnp.where(ids_t < SPLIT, 2 * ids_t, 2 * (ids_t - SPLIT) + 1)
    idx2d = ids_r.reshape(ntok // IDX_MINOR, IDX_MINOR)

    g = _gather_scatter_sc(pv, idx2d, ntok, batch)
    g128 = g.reshape(ntok // 2, 2 * HIDDEN)

    out_t = _unpack_transpose_tc(g128, hist, batch)
    return jnp.transpose(out_t, (2, 0, 1))


# speedup vs baseline: 2.6343x; 1.0699x over previous
import functools

import jax
import jax.numpy as jnp
from jax import lax
from jax.experimental import pallas as pl
from jax.experimental.pallas import tpu as pltpu
from jax.experimental.pallas import tpu_sc as plsc

HIDDEN = 64
NUM_CORES = 2
NUM_SUBCORES = 16
NW = NUM_CORES * NUM_SUBCORES

IDX_MINOR = 128
ROWS_PER_DMA = 128
DMAS_PER_CHUNK = 5
CHUNK_TOK = ROWS_PER_DMA * DMAS_PER_CHUNK

VB = 16384
NHI = 30
SPLIT = NHI * VB

NSEG = 4


def _project_pack_tc(tt, w, b2d, vocab):
    rows = vocab - SPLIT
    grid = pl.cdiv(rows, VB)

    def body(lo_ref, hi_ref, w_ref, b_ref, o_ref):
        wv = w_ref[...]
        bv = b_ref[...]
        o_ref[:, 0:HIDDEN] = lax.dot_general(
            lo_ref[...], wv, (((0,), (0,)), ((), ())),
            preferred_element_type=jnp.float32) + bv
        o_ref[:, HIDDEN:2 * HIDDEN] = lax.dot_general(
            hi_ref[...], wv, (((0,), (0,)), ((), ())),
            preferred_element_type=jnp.float32) + bv

    return pl.pallas_call(
        body,
        grid=(grid,),
        in_specs=[
            pl.BlockSpec((HIDDEN, VB), lambda i: (0, i)),
            pl.BlockSpec((HIDDEN, VB), lambda i: (0, i + NHI)),
            pl.BlockSpec((HIDDEN, HIDDEN), lambda i: (0, 0)),
            pl.BlockSpec((1, HIDDEN), lambda i: (0, 0)),
        ],
        out_specs=pl.BlockSpec((VB, 2 * HIDDEN), lambda i: (i, 0)),
        out_shape=jax.ShapeDtypeStruct((rows, 2 * HIDDEN), jnp.float32),
    )(tt, tt, w, b2d)


def _gather_scatter_sc(table, idx2d, seg, ntok_c, batch):
    tok_per_w = ntok_c // NW
    rows_per_w = tok_per_w // IDX_MINOR
    chunks = tok_per_w // CHUNK_TOK
    halfb = batch // 2
    seg_rows = seg * (ntok_c // IDX_MINOR)

    mesh = plsc.VectorSubcoreMesh(core_axis_name="c", subcore_axis_name="s")

    @functools.partial(
        pl.kernel,
        mesh=mesh,
        out_type=jax.ShapeDtypeStruct((ntok_c, HIDDEN), jnp.float32),
        compiler_params=pltpu.CompilerParams(use_tc_tiling_on_sc=False),
        scratch_types=[
            pltpu.VMEM((rows_per_w, IDX_MINOR), jnp.int32),
            pltpu.VMEM((CHUNK_TOK, HIDDEN), jnp.float32),
            pltpu.VMEM((CHUNK_TOK, HIDDEN), jnp.float32),
            pltpu.VMEM((DMAS_PER_CHUNK, IDX_MINOR), jnp.int32),
            pltpu.VMEM((DMAS_PER_CHUNK, IDX_MINOR), jnp.int32),
            pltpu.SemaphoreType.DMA,
            pltpu.SemaphoreType.DMA,
            pltpu.SemaphoreType.DMA,
            pltpu.SemaphoreType.DMA,
        ],
    )
    def k(tbl, idxh, outh, idx_v, buf0, buf1, dst0, dst1,
          gsem0, gsem1, wsem0, wsem1):
        wid = lax.axis_index("s") * NUM_CORES + lax.axis_index("c")
        rbase = seg_rows + wid * rows_per_w
        tbase = wid * tok_per_w
        bufs = (buf0, buf1)
        dsts = (dst0, dst1)
        gsems = (gsem0, gsem1)
        wsems = (wsem0, wsem1)

        pltpu.sync_copy(idxh.at[pl.ds(rbase, rows_per_w)], idx_v)

        iota16 = lax.iota(jnp.int32, 16)

        def g_desc(g, j, p):
            return pltpu.make_async_copy(
                tbl.at[idx_v.at[g * DMAS_PER_CHUNK + j]],
                bufs[p].at[pl.ds(j * ROWS_PER_DMA, ROWS_PER_DMA)],
                gsems[p])

        def w_desc(j, p):
            return pltpu.make_async_copy(
                bufs[p].at[pl.ds(j * ROWS_PER_DMA, ROWS_PER_DMA)],
                outh.at[dsts[p].at[j]],
                wsems[p])

        def fire(g, p):
            for j in range(DMAS_PER_CHUNK):
                g_desc(g, j, p).start()

        def wait_g(g, p):
            for j in range(DMAS_PER_CHUNK):
                g_desc(g, j, p).wait()

        def wb_start(g, p):
            for j in range(DMAS_PER_CHUNK):
                t0 = tbase + g * CHUNK_TOK + j * ROWS_PER_DMA
                m = lax.rem(t0, batch)
                base = (t0 - m) + 2 * lax.rem(m, halfb) + lax.div(m, halfb)
                for q in range(IDX_MINOR // 16):
                    dsts[p][j, pl.ds(q * 16, 16)] = (
                        iota16 * 2 + (base + 32 * q))
            for j in range(DMAS_PER_CHUNK):
                w_desc(j, p).start()

        def wb_wait(p):
            for j in range(DMAS_PER_CHUNK):
                w_desc(j, p).wait()

        fire(0, 0)

        def body(t, carry):
            i = 2 * t

            @pl.when(i > 0)
            def _():
                wb_wait(1)

            fire(i + 1, 1)
            wait_g(i, 0)
            wb_start(i, 0)
            wait_g(i + 1, 1)
            wb_start(i + 1, 1)

            @pl.when(i + 2 < chunks)
            def _():
                wb_wait(0)
                fire(i + 2, 0)

            return carry

        lax.fori_loop(0, chunks // 2, body, 0)
        wb_wait(0)
        wb_wait(1)

    return k(table, idx2d)


def _unpack_transpose_tc(g128, seg, lseg, hist, batch, prev=None):
    cb = batch // 2

    def body(x_ref, *refs):
        o_ref = refs[-1]
        x = x_ref[...]
        hh = lax.broadcasted_iota(jnp.int32, (HIDDEN, 2 * HIDDEN), 0)
        mm = lax.broadcasted_iota(jnp.int32, (HIDDEN, 2 * HIDDEN), 1)
        i_lo = (mm == hh).astype(jnp.float32)
        i_hi = (mm == hh + HIDDEN).astype(jnp.float32)
        o_ref[0, :, 0:cb] = lax.dot_general(
            i_lo, x, (((1,), (1,)), ((), ())),
            preferred_element_type=jnp.float32)
        o_ref[0, :, cb:batch] = lax.dot_general(
            i_hi, x, (((1,), (1,)), ((), ())),
            preferred_element_type=jnp.float32)

    in_specs = [pl.BlockSpec((cb, 2 * HIDDEN), lambda i: (i, 0))]
    args = [g128]
    aliases = {}
    if prev is not None:
        in_specs.append(pl.BlockSpec(memory_space=pl.ANY))
        args.append(prev)
        aliases = {1: 0}

    return pl.pallas_call(
        body,
        grid=(lseg,),
        in_specs=in_specs,
        out_specs=pl.BlockSpec((1, HIDDEN, batch),
                               lambda i, s=seg, n=lseg: (i + s * n, 0, 0)),
        out_shape=jax.ShapeDtypeStruct((hist, HIDDEN, batch), jnp.float32),
        input_output_aliases=aliases,
    )(*args)


def kernel(input_ids, emb_table, W, b):
    batch, hist = input_ids.shape
    vocab = emb_table.shape[0]
    ntok = batch * hist
    ntok_c = ntok // NSEG
    lseg = hist // NSEG

    tt = jnp.transpose(emb_table)
    p2 = _project_pack_tc(tt, W, b.reshape(1, HIDDEN), vocab)
    pv = p2.reshape(2 * (vocab - SPLIT), HIDDEN)

    ids_t = jnp.transpose(input_ids).astype(jnp.int32)
    ids_r = jnp.where(ids_t < SPLIT, 2 * ids_t, 2 * (ids_t - SPLIT) + 1)
    idx2d = ids_r.reshape(ntok // IDX_MINOR, IDX_MINOR)

    out_t = None
    for seg in range(NSEG):
        g = _gather_scatter_sc(pv, idx2d, seg, ntok_c, batch)
        g128 = g.reshape(ntok_c // 2, 2 * HIDDEN)
        out_t = _unpack_transpose_tc(g128, seg, lseg, hist, batch, out_t)

    return jnp.transpose(out_t, (2, 0, 1))


# speedup vs baseline: 2.8319x; 1.0750x over previous
import functools

import jax
import jax.numpy as jnp
from jax import lax
from jax.experimental import pallas as pl
from jax.experimental.pallas import tpu as pltpu
from jax.experimental.pallas import tpu_sc as plsc

HIDDEN = 64
NUM_CORES = 2
NUM_SUBCORES = 16
NW = NUM_CORES * NUM_SUBCORES

IDX_MINOR = 128
ROWS_PER_DMA = 128
DMAS_PER_CHUNK = 5
CHUNK_TOK = ROWS_PER_DMA * DMAS_PER_CHUNK

VB = 16384
NHI = 30
SPLIT = NHI * VB

NSEG = 4


def _project_pack_tc(tt, w, b2d, vocab):
    rows = vocab - SPLIT
    grid = pl.cdiv(rows, VB)

    def body(lo_ref, hi_ref, w_ref, b_ref, o_ref):
        wv = w_ref[...]
        bv = b_ref[...]
        o_ref[:, 0:HIDDEN] = lax.dot_general(
            lo_ref[...], wv, (((0,), (0,)), ((), ())),
            preferred_element_type=jnp.float32) + bv
        o_ref[:, HIDDEN:2 * HIDDEN] = lax.dot_general(
            hi_ref[...], wv, (((0,), (0,)), ((), ())),
            preferred_element_type=jnp.float32) + bv

    return pl.pallas_call(
        body,
        grid=(grid,),
        in_specs=[
            pl.BlockSpec((HIDDEN, VB), lambda i: (0, i)),
            pl.BlockSpec((HIDDEN, VB), lambda i: (0, i + NHI)),
            pl.BlockSpec((HIDDEN, HIDDEN), lambda i: (0, 0)),
            pl.BlockSpec((1, HIDDEN), lambda i: (0, 0)),
        ],
        out_specs=pl.BlockSpec((VB, 2 * HIDDEN), lambda i: (i, 0)),
        out_shape=jax.ShapeDtypeStruct((rows, 2 * HIDDEN), jnp.float32),
    )(tt, tt, w, b2d)


def _gather_scatter_sc(table, idx2d, seg, ntok_c, batch):
    tok_per_w = ntok_c // NW
    rows_per_w = tok_per_w // IDX_MINOR
    chunks = tok_per_w // CHUNK_TOK
    halfb = batch // 2
    seg_rows = seg * (ntok_c // IDX_MINOR)

    mesh = plsc.VectorSubcoreMesh(core_axis_name="c", subcore_axis_name="s")

    @functools.partial(
        pl.kernel,
        mesh=mesh,
        out_type=jax.ShapeDtypeStruct((ntok_c, HIDDEN), jnp.float32),
        compiler_params=pltpu.CompilerParams(use_tc_tiling_on_sc=False),
        scratch_types=[
            pltpu.VMEM((rows_per_w, IDX_MINOR), jnp.int32),
            pltpu.VMEM((CHUNK_TOK, HIDDEN), jnp.float32),
            pltpu.VMEM((CHUNK_TOK, HIDDEN), jnp.float32),
            pltpu.VMEM((DMAS_PER_CHUNK, IDX_MINOR), jnp.int32),
            pltpu.VMEM((DMAS_PER_CHUNK, IDX_MINOR), jnp.int32),
            pltpu.SemaphoreType.DMA,
            pltpu.SemaphoreType.DMA,
            pltpu.SemaphoreType.DMA,
            pltpu.SemaphoreType.DMA,
        ],
    )
    def k(tbl, idxh, outh, idx_v, buf0, buf1, dst0, dst1,
          gsem0, gsem1, wsem0, wsem1):
        wid = lax.axis_index("s") * NUM_CORES + lax.axis_index("c")
        rbase = seg_rows + wid * rows_per_w
        tbase = wid * tok_per_w
        bufs = (buf0, buf1)
        dsts = (dst0, dst1)
        gsems = (gsem0, gsem1)
        wsems = (wsem0, wsem1)

        pltpu.sync_copy(idxh.at[pl.ds(rbase, rows_per_w)], idx_v)

        iota16 = lax.iota(jnp.int32, 16)

        def g_desc(g, j, p):
            return pltpu.make_async_copy(
                tbl.at[idx_v.at[g * DMAS_PER_CHUNK + j]],
                bufs[p].at[pl.ds(j * ROWS_PER_DMA, ROWS_PER_DMA)],
                gsems[p])

        def w_desc(j, p):
            return pltpu.make_async_copy(
                bufs[p].at[pl.ds(j * ROWS_PER_DMA, ROWS_PER_DMA)],
                outh.at[dsts[p].at[j]],
                wsems[p])

        def fire(g, p):
            for j in range(DMAS_PER_CHUNK):
                g_desc(g, j, p).start()

        def wait_g(g, p):
            for j in range(DMAS_PER_CHUNK):
                g_desc(g, j, p).wait()

        def wb_start(g, p):
            for j in range(DMAS_PER_CHUNK):
                t0 = tbase + g * CHUNK_TOK + j * ROWS_PER_DMA
                m = lax.rem(t0, batch)
                base = (t0 - m) + 2 * lax.rem(m, halfb) + lax.div(m, halfb)
                for q in range(IDX_MINOR // 16):
                    dsts[p][j, pl.ds(q * 16, 16)] = (
                        iota16 * 2 + (base + 32 * q))
            for j in range(DMAS_PER_CHUNK):
                w_desc(j, p).start()

        def wb_wait(p):
            for j in range(DMAS_PER_CHUNK):
                w_desc(j, p).wait()

        fire(0, 0)

        def body(t, carry):
            i = 2 * t

            @pl.when(i > 0)
            def _():
                wb_wait(1)

            fire(i + 1, 1)
            wait_g(i, 0)
            wb_start(i, 0)
            wait_g(i + 1, 1)
            wb_start(i + 1, 1)

            @pl.when(i + 2 < chunks)
            def _():
                wb_wait(0)
                fire(i + 2, 0)

            return carry

        lax.fori_loop(0, chunks // 2, body, 0)
        wb_wait(0)
        wb_wait(1)

    return k(table, idx2d)


def _unpack_transpose_tc(g128, seg, lseg, hist, batch, prev=None):
    cb = batch // 2
    lb = 5

    def body(x_ref, *refs):
        o_ref = refs[-1]
        hh = lax.broadcasted_iota(jnp.int32, (HIDDEN, 2 * HIDDEN), 0)
        mm = lax.broadcasted_iota(jnp.int32, (HIDDEN, 2 * HIDDEN), 1)
        i_lo = (mm == hh).astype(jnp.float32)
        i_hi = (mm == hh + HIDDEN).astype(jnp.float32)
        for j in range(lb):
            x = x_ref[pl.ds(j * cb, cb), :]
            o_ref[j, :, 0:cb] = lax.dot_general(
                i_lo, x, (((1,), (1,)), ((), ())),
                preferred_element_type=jnp.float32)
            o_ref[j, :, cb:batch] = lax.dot_general(
                i_hi, x, (((1,), (1,)), ((), ())),
                preferred_element_type=jnp.float32)

    in_specs = [pl.BlockSpec((lb * cb, 2 * HIDDEN), lambda i: (i, 0))]
    args = [g128]
    aliases = {}
    if prev is not None:
        in_specs.append(pl.BlockSpec(memory_space=pl.ANY))
        args.append(prev)
        aliases = {1: 0}

    return pl.pallas_call(
        body,
        grid=(lseg // lb,),
        in_specs=in_specs,
        out_specs=pl.BlockSpec((lb, HIDDEN, batch),
                               lambda i, s=seg, n=lseg // lb: (i + s * n, 0, 0)),
        out_shape=jax.ShapeDtypeStruct((hist, HIDDEN, batch), jnp.float32),
        input_output_aliases=aliases,
    )(*args)


def kernel(input_ids, emb_table, W, b):
    batch, hist = input_ids.shape
    vocab = emb_table.shape[0]
    ntok = batch * hist
    ntok_c = ntok // NSEG
    lseg = hist // NSEG

    tt = jnp.transpose(emb_table)
    p2 = _project_pack_tc(tt, W, b.reshape(1, HIDDEN), vocab)
    pv = p2.reshape(2 * (vocab - SPLIT), HIDDEN)

    ids_t = jnp.transpose(input_ids).astype(jnp.int32)
    ids_r = jnp.where(ids_t < SPLIT, 2 * ids_t, 2 * (ids_t - SPLIT) + 1)
    idx2d = ids_r.reshape(ntok // IDX_MINOR, IDX_MINOR)

    out_t = None
    for seg in range(NSEG):
        g = _gather_scatter_sc(pv, idx2d, seg, ntok_c, batch)
        g128 = g.reshape(ntok_c // 2, 2 * HIDDEN)
        out_t = _unpack_transpose_tc(g128, seg, lseg, hist, batch, out_t)

    return jnp.transpose(out_t, (2, 0, 1))


# speedup vs baseline: 3.0441x; 1.0749x over previous
import functools

import jax
import jax.numpy as jnp
from jax import lax
from jax.experimental import pallas as pl
from jax.experimental.pallas import tpu as pltpu
from jax.experimental.pallas import tpu_sc as plsc

HIDDEN = 64
NUM_CORES = 2
NUM_SUBCORES = 16
NW = NUM_CORES * NUM_SUBCORES

IDX_MINOR = 128
ROWS_PER_DMA = 128
DMAS_PER_CHUNK = 5
CHUNK_TOK = ROWS_PER_DMA * DMAS_PER_CHUNK

VB = 16384
NHI = 30
SPLIT = NHI * VB

NSEG = 4


def _project_pack_tc(tt, w, b2d, vocab):
    rows = vocab - SPLIT
    grid = pl.cdiv(rows, VB)

    def body(lo_ref, hi_ref, w_ref, b_ref, o_ref):
        wv = w_ref[...].astype(jnp.bfloat16)
        bv = b_ref[...]
        o_ref[:, 0:HIDDEN] = lax.dot_general(
            lo_ref[...].astype(jnp.bfloat16), wv, (((0,), (0,)), ((), ())),
            preferred_element_type=jnp.float32) + bv
        o_ref[:, HIDDEN:2 * HIDDEN] = lax.dot_general(
            hi_ref[...].astype(jnp.bfloat16), wv, (((0,), (0,)), ((), ())),
            preferred_element_type=jnp.float32) + bv

    return pl.pallas_call(
        body,
        grid=(grid,),
        in_specs=[
            pl.BlockSpec((HIDDEN, VB), lambda i: (0, i)),
            pl.BlockSpec((HIDDEN, VB), lambda i: (0, i + NHI)),
            pl.BlockSpec((HIDDEN, HIDDEN), lambda i: (0, 0)),
            pl.BlockSpec((1, HIDDEN), lambda i: (0, 0)),
        ],
        out_specs=pl.BlockSpec((VB, 2 * HIDDEN), lambda i: (i, 0)),
        out_shape=jax.ShapeDtypeStruct((rows, 2 * HIDDEN), jnp.float32),
    )(tt, tt, w, b2d)


def _gather_scatter_sc(table, idx2d, seg, ntok_c, batch):
    tok_per_w = ntok_c // NW
    rows_per_w = tok_per_w // IDX_MINOR
    chunks = tok_per_w // CHUNK_TOK
    halfb = batch // 2
    seg_rows = seg * (ntok_c // IDX_MINOR)

    mesh = plsc.VectorSubcoreMesh(core_axis_name="c", subcore_axis_name="s")

    @functools.partial(
        pl.kernel,
        mesh=mesh,
        out_type=jax.ShapeDtypeStruct((ntok_c, HIDDEN), jnp.float32),
        compiler_params=pltpu.CompilerParams(use_tc_tiling_on_sc=False),
        scratch_types=[
            pltpu.VMEM((rows_per_w, IDX_MINOR), jnp.int32),
            pltpu.VMEM((CHUNK_TOK, HIDDEN), jnp.float32),
            pltpu.VMEM((CHUNK_TOK, HIDDEN), jnp.float32),
            pltpu.VMEM((DMAS_PER_CHUNK, IDX_MINOR), jnp.int32),
            pltpu.VMEM((DMAS_PER_CHUNK, IDX_MINOR), jnp.int32),
            pltpu.SemaphoreType.DMA,
            pltpu.SemaphoreType.DMA,
            pltpu.SemaphoreType.DMA,
            pltpu.SemaphoreType.DMA,
        ],
    )
    def k(tbl, idxh, outh, idx_v, buf0, buf1, dst0, dst1,
          gsem0, gsem1, wsem0, wsem1):
        wid = lax.axis_index("s") * NUM_CORES + lax.axis_index("c")
        rbase = seg_rows + wid * rows_per_w
        tbase = wid * tok_per_w
        bufs = (buf0, buf1)
        dsts = (dst0, dst1)
        gsems = (gsem0, gsem1)
        wsems = (wsem0, wsem1)

        pltpu.sync_copy(idxh.at[pl.ds(rbase, rows_per_w)], idx_v)

        iota16 = lax.iota(jnp.int32, 16)

        def g_desc(g, j, p):
            return pltpu.make_async_copy(
                tbl.at[idx_v.at[g * DMAS_PER_CHUNK + j]],
                bufs[p].at[pl.ds(j * ROWS_PER_DMA, ROWS_PER_DMA)],
                gsems[p])

        def w_desc(j, p):
            return pltpu.make_async_copy(
                bufs[p].at[pl.ds(j * ROWS_PER_DMA, ROWS_PER_DMA)],
                outh.at[dsts[p].at[j]],
                wsems[p])

        def fire(g, p):
            for j in range(DMAS_PER_CHUNK):
                g_desc(g, j, p).start()

        def wait_g(g, p):
            for j in range(DMAS_PER_CHUNK):
                g_desc(g, j, p).wait()

        def wb_start(g, p):
            for j in range(DMAS_PER_CHUNK):
                t0 = tbase + g * CHUNK_TOK + j * ROWS_PER_DMA
                m = lax.rem(t0, batch)
                base = (t0 - m) + 2 * lax.rem(m, halfb) + lax.div(m, halfb)
                for q in range(IDX_MINOR // 16):
                    dsts[p][j, pl.ds(q * 16, 16)] = (
                        iota16 * 2 + (base + 32 * q))
            for j in range(DMAS_PER_CHUNK):
                w_desc(j, p).start()

        def wb_wait(p):
            for j in range(DMAS_PER_CHUNK):
                w_desc(j, p).wait()

        fire(0, 0)

        def body(t, carry):
            i = 2 * t

            @pl.when(i > 0)
            def _():
                wb_wait(1)

            fire(i + 1, 1)
            wait_g(i, 0)
            wb_start(i, 0)
            wait_g(i + 1, 1)
            wb_start(i + 1, 1)

            @pl.when(i + 2 < chunks)
            def _():
                wb_wait(0)
                fire(i + 2, 0)

            return carry

        lax.fori_loop(0, chunks // 2, body, 0)
        wb_wait(0)
        wb_wait(1)

    return k(table, idx2d)


def _unpack_transpose_tc(g128, seg, lseg, hist, batch, prev=None):
    cb = batch // 2
    lb = 5

    def body(x_ref, *refs):
        o_ref = refs[-1]
        hh = lax.broadcasted_iota(jnp.int32, (HIDDEN, 2 * HIDDEN), 0)
        mm = lax.broadcasted_iota(jnp.int32, (HIDDEN, 2 * HIDDEN), 1)
        i_lo = (mm == hh).astype(jnp.float32)
        i_hi = (mm == hh + HIDDEN).astype(jnp.float32)
        for j in range(lb):
            x = x_ref[pl.ds(j * cb, cb), :]
            o_ref[j, :, 0:cb] = lax.dot_general(
                i_lo, x, (((1,), (1,)), ((), ())),
                preferred_element_type=jnp.float32)
            o_ref[j, :, cb:batch] = lax.dot_general(
                i_hi, x, (((1,), (1,)), ((), ())),
                preferred_element_type=jnp.float32)

    in_specs = [pl.BlockSpec((lb * cb, 2 * HIDDEN), lambda i: (i, 0))]
    args = [g128]
    aliases = {}
    if prev is not None:
        in_specs.append(pl.BlockSpec(memory_space=pl.ANY))
        args.append(prev)
        aliases = {1: 0}

    return pl.pallas_call(
        body,
        grid=(lseg // lb,),
        in_specs=in_specs,
        out_specs=pl.BlockSpec((lb, HIDDEN, batch),
                               lambda i, s=seg, n=lseg // lb: (i + s * n, 0, 0)),
        out_shape=jax.ShapeDtypeStruct((hist, HIDDEN, batch), jnp.float32),
        input_output_aliases=aliases,
    )(*args)


def kernel(input_ids, emb_table, W, b):
    batch, hist = input_ids.shape
    vocab = emb_table.shape[0]
    ntok = batch * hist
    ntok_c = ntok // NSEG
    lseg = hist // NSEG

    tt = jnp.transpose(emb_table)
    p2 = _project_pack_tc(tt, W, b.reshape(1, HIDDEN), vocab)
    pv = p2.reshape(2 * (vocab - SPLIT), HIDDEN)

    ids_t = jnp.transpose(input_ids).astype(jnp.int32)
    ids_r = jnp.where(ids_t < SPLIT, 2 * ids_t, 2 * (ids_t - SPLIT) + 1)
    idx2d = ids_r.reshape(ntok // IDX_MINOR, IDX_MINOR)

    out_t = None
    for seg in range(NSEG):
        g = _gather_scatter_sc(pv, idx2d, seg, ntok_c, batch)
        g128 = g.reshape(ntok_c // 2, 2 * HIDDEN)
        out_t = _unpack_transpose_tc(g128, seg, lseg, hist, batch, out_t)

    return jnp.transpose(out_t, (2, 0, 1))
